# baseline (speedup 1.0000x reference)
import functools

import jax
import jax.numpy as jnp
from jax import lax
from jax.experimental import pallas as pl
from jax.experimental.pallas import tpu as pltpu
from jax.experimental.pallas import tpu_sc as plsc

N = 10000
E = 320000
D = 64
DIN = 128
DOUT = 128
NC, NS = 2, 16
NW = NC * NS
EPW = E // NW
CH = 125
NCHUNK = EPW // CH
NB = 8
NG = NCHUNK // NB
NP = 10240
RPT = NP // NS
NE16 = EPW // 16
NH = NP // 2
RH = 1024

_sc_mesh = plsc.VectorSubcoreMesh(core_axis_name="c", subcore_axis_name="s")


@functools.partial(
    pl.kernel,
    out_type=jax.ShapeDtypeStruct((NW, NP), jnp.float32),
    mesh=_sc_mesh,
    compiler_params=pltpu.CompilerParams(use_tc_tiling_on_sc=False,
                                         needs_layout_passes=False),
    scratch_types=[
        pltpu.VMEM((NE16, 16), jnp.int32),
        pltpu.VMEM((NP,), jnp.float32),
    ],
)
def _sc_degree(dst_hbm, out_hbm, dst_v, tacc):
    cid = lax.axis_index("c")
    sid = lax.axis_index("s")
    wid = sid * NC + cid
    pltpu.sync_copy(dst_hbm.at[wid], dst_v)

    def zfill(i, _):
        tacc[pl.ds(i * 16, 16)] = jnp.zeros((16,), jnp.float32)
        return _

    lax.fori_loop(0, NP // 16, zfill, None)

    def count(i, _):
        idx = dst_v[i]
        cnt, last = plsc.scan_count(idx)
        plsc.addupdate_scatter(tacc, [idx], cnt.astype(jnp.float32),
                               mask=last)
        return _

    lax.fori_loop(0, NE16, count, None)
    pltpu.sync_copy(tacc, out_hbm.at[wid])


@functools.partial(
    pl.kernel,
    out_type=jax.ShapeDtypeStruct((NC, NP, D), jnp.float32),
    mesh=_sc_mesh,
    compiler_params=pltpu.CompilerParams(use_tc_tiling_on_sc=False),
    scratch_types=[
        pltpu.VMEM((NCHUNK, CH), jnp.int32),
        pltpu.VMEM((NCHUNK, CH), jnp.int32),
        pltpu.VMEM((NB, CH, D), jnp.float32),
        pltpu.VMEM((80, D), jnp.float32),
        pltpu.SemaphoreType.DMA((NB,)),
        pltpu.SemaphoreType.DMA((NB,)),
        pltpu.SemaphoreType.DMA,
        pltpu.VMEM_SHARED((NP, D), jnp.float32),
    ],
)
def _sc_edge_pass(g_hbm, src_hbm, dst_hbm, out_hbm, src_v, dst_v, rows_v,
                  z_v, gsem, ssem, zsem, acc):
    cid = lax.axis_index("c")
    sid = lax.axis_index("s")
    wid = sid * NC + cid
    pltpu.sync_copy(src_hbm.at[wid], src_v)
    pltpu.sync_copy(dst_hbm.at[wid], dst_v)

    for b in range(NB):
        pltpu.async_copy(g_hbm.at[src_v.at[b]], rows_v.at[b], gsem.at[b])

    def zfill(i, _):
        for j in range(D // 16):
            z_v[i, pl.ds(j * 16, 16)] = jnp.zeros((16,), jnp.float32)
        return _

    lax.fori_loop(0, 80, zfill, None)
    for c in range(RPT // 80):
        pltpu.async_copy(z_v, acc.at[pl.ds(sid * RPT + c * 80, 80)], zsem)
    for c in range(RPT // 80):
        pltpu.make_async_copy(z_v, acc.at[pl.ds(sid * RPT, 80)], zsem).wait()
    plsc.subcore_barrier()

    def group(g, _):
        for b in range(NB):
            k = g * NB + b
            pltpu.make_async_copy(
                g_hbm.at[src_v.at[k]], rows_v.at[b], gsem.at[b]).wait()
            pltpu.async_copy(
                rows_v.at[b], acc.at[dst_v.at[k]], ssem.at[b], add=True)
            nk = k + NB

            @pl.when(nk < NCHUNK)
            def _start():
                pltpu.make_async_copy(
                    rows_v.at[b], acc.at[dst_v.at[k]], ssem.at[b]).wait()
                pltpu.async_copy(
                    g_hbm.at[src_v.at[nk]], rows_v.at[b], gsem.at[b])

        return _

    lax.fori_loop(0, NG, group, None)
    for b in range(NB):
        pltpu.make_async_copy(
            rows_v.at[b], acc.at[dst_v.at[0]], ssem.at[b]).wait()
    plsc.subcore_barrier()
    pltpu.sync_copy(acc.at[pl.ds(sid * RPT, RPT)],
                    out_hbm.at[cid, pl.ds(sid * RPT, RPT)])


_R = 2048


def _tc_matmul(x, w1):
    def body(x_ref, w_ref, u_ref):
        xr = x_ref[...].reshape(RH, 2, DIN)
        ue = jnp.dot(xr[:, 0, :], w_ref[...],
                     preferred_element_type=jnp.float32)
        uo = jnp.dot(xr[:, 1, :], w_ref[...],
                     preferred_element_type=jnp.float32)
        u_ref[...] = jnp.concatenate([ue, uo], axis=1)

    return pl.pallas_call(
        body,
        grid=(NP // _R,),
        in_specs=[
            pl.BlockSpec((_R, DIN), lambda i: (i, 0)),
            pl.BlockSpec((DIN, D), lambda i: (0, 0)),
        ],
        out_specs=pl.BlockSpec((RH, 2 * D), lambda i: (i, 0)),
        out_shape=jax.ShapeDtypeStruct((NH, 2 * D), jnp.float32),
    )(x, w1)


def _tc_mult(degp):
    def body(dp_ref, m_ref):
        acc = dp_ref[0]
        for j in range(1, NW):
            acc = acc + dp_ref[j]
        me = lax.rsqrt(acc[:, 0:1] + 1.0)
        mo = lax.rsqrt(acc[:, 1:2] + 1.0)
        m_ref[...] = jnp.concatenate(
            [jnp.broadcast_to(me, (RH, D)), jnp.broadcast_to(mo, (RH, D))],
            axis=1)

    return pl.pallas_call(
        body,
        grid=(NH // RH,),
        in_specs=[pl.BlockSpec((NW, RH, 2), lambda i: (0, i, 0))],
        out_specs=pl.BlockSpec((RH, 2 * D), lambda i: (i, 0)),
        out_shape=jax.ShapeDtypeStruct((NH, 2 * D), jnp.float32),
    )(degp.reshape(NW, NH, 2))


def _tc_scale(u, m):
    def body(u_ref, m_ref, g_ref):
        g_ref[...] = m_ref[...] * u_ref[...]

    return pl.pallas_call(
        body,
        grid=(NH // RH,),
        in_specs=[
            pl.BlockSpec((RH, 2 * D), lambda i: (i, 0)),
            pl.BlockSpec((RH, 2 * D), lambda i: (i, 0)),
        ],
        out_specs=pl.BlockSpec((RH, 2 * D), lambda i: (i, 0)),
        out_shape=jax.ShapeDtypeStruct((NH, 2 * D), jnp.float32),
    )(u, m)


def _tc_mid(s, g, m, wbd, bp):
    def body(s_ref, g_ref, m_ref, w_ref, b_ref, o_ref):
        mm = m_ref[...]
        h = jnp.maximum(mm * (s_ref[0] + s_ref[1] + g_ref[...]) + b_ref[...],
                        0.0)
        o_ref[...] = mm * jnp.dot(h, w_ref[...],
                                  preferred_element_type=jnp.float32)

    return pl.pallas_call(
        body,
        grid=(NH // RH,),
        in_specs=[
            pl.BlockSpec((NC, RH, 2 * D), lambda i: (0, i, 0)),
            pl.BlockSpec((RH, 2 * D), lambda i: (i, 0)),
            pl.BlockSpec((RH, 2 * D), lambda i: (i, 0)),
            pl.BlockSpec((2 * D, 2 * D), lambda i: (0, 0)),
            pl.BlockSpec((1, 2 * D), lambda i: (0, 0)),
        ],
        out_specs=pl.BlockSpec((RH, 2 * D), lambda i: (i, 0)),
        out_shape=jax.ShapeDtypeStruct((NH, 2 * D), jnp.float32),
    )(s, g, m, wbd, bp)


def _tc_out(s, q, m, w2e, w2o, b2):
    def body(s_ref, q_ref, m_ref, we_ref, wo_ref, b_ref, o_ref):
        a = m_ref[...] * (s_ref[0] + s_ref[1] + q_ref[...])
        he = jnp.dot(a, we_ref[...], preferred_element_type=jnp.float32)
        ho = jnp.dot(a, wo_ref[...], preferred_element_type=jnp.float32)
        st = jnp.concatenate(
            [he.reshape(RH, 1, DOUT), ho.reshape(RH, 1, DOUT)], axis=1)
        o_ref[...] = jnp.maximum(st.reshape(2 * RH, DOUT) + b_ref[...], 0.0)

    return pl.pallas_call(
        body,
        grid=(NP // _R,),
        in_specs=[
            pl.BlockSpec((NC, RH, 2 * D), lambda i: (0, i, 0)),
            pl.BlockSpec((RH, 2 * D), lambda i: (i, 0)),
            pl.BlockSpec((RH, 2 * D), lambda i: (i, 0)),
            pl.BlockSpec((2 * D, DOUT), lambda i: (0, 0)),
            pl.BlockSpec((2 * D, DOUT), lambda i: (0, 0)),
            pl.BlockSpec((1, DOUT), lambda i: (0, 0)),
        ],
        out_specs=pl.BlockSpec((_R, DOUT), lambda i: (i, 0)),
        out_shape=jax.ShapeDtypeStruct((N, DOUT), jnp.float32),
    )(s, q, m, w2e, w2o, b2)


def kernel(x, edge_index, W1, b1, W3, b3, W2, b2):
    src = edge_index[0].reshape(NW, NCHUNK, CH)
    dst = edge_index[1].reshape(NW, NCHUNK, CH)
    u1 = _tc_matmul(x, W1)
    degp = _sc_degree(edge_index[1].reshape(NW, NE16, 16))
    m = _tc_mult(degp)
    g1 = _tc_scale(u1, m)

    z = jnp.zeros_like(W3)
    w3bd = jnp.block([[W3, z], [z, W3]])
    eye = jnp.eye(2 * D, dtype=jnp.float32)
    w2e = jnp.concatenate([W2, jnp.zeros_like(W2)], axis=0)
    w2o = jnp.concatenate([jnp.zeros_like(W2), W2], axis=0)
    bp1 = jnp.concatenate([b1, b1]).reshape(1, 2 * D)
    bp3 = jnp.concatenate([b3, b3]).reshape(1, 2 * D)

    def sc(gp):
        out = _sc_edge_pass(gp.reshape(NP, D), src, dst)
        return out.reshape(NC, NH, 2 * D)

    s1 = sc(g1)
    g2 = _tc_mid(s1, g1, m, w3bd, bp1)
    s2 = sc(g2)
    q = _tc_mid(s2, g2, m, eye, bp3)
    s3 = sc(q)
    return _tc_out(s3, q, m, w2e, w2o, b2.reshape(1, DOUT))


# speedup vs baseline: 1.0212x; 1.0212x over previous
import functools

import jax
import jax.numpy as jnp
from jax import lax
from jax.experimental import pallas as pl
from jax.experimental.pallas import tpu as pltpu
from jax.experimental.pallas import tpu_sc as plsc

N = 10000
E = 320000
D = 64
DIN = 128
DOUT = 128
NC, NS = 2, 16
NW = NC * NS
EPW = E // NW
CH = 100
NCHUNK = EPW // CH
NB = 10
NG = NCHUNK // NB
NP = 10240
RPT = NP // NS
NE16 = EPW // 16
NH = NP // 2
RH = 1024

_sc_mesh = plsc.VectorSubcoreMesh(core_axis_name="c", subcore_axis_name="s")


@functools.partial(
    pl.kernel,
    out_type=jax.ShapeDtypeStruct((NW, NP), jnp.float32),
    mesh=_sc_mesh,
    compiler_params=pltpu.CompilerParams(use_tc_tiling_on_sc=False,
                                         needs_layout_passes=False),
    scratch_types=[
        pltpu.VMEM((NE16, 16), jnp.int32),
        pltpu.VMEM((NP,), jnp.float32),
    ],
)
def _sc_degree(dst_hbm, out_hbm, dst_v, tacc):
    cid = lax.axis_index("c")
    sid = lax.axis_index("s")
    wid = sid * NC + cid
    pltpu.sync_copy(dst_hbm.at[wid], dst_v)

    def zfill(i, _):
        tacc[pl.ds(i * 16, 16)] = jnp.zeros((16,), jnp.float32)
        return _

    lax.fori_loop(0, NP // 16, zfill, None)

    def count(i, _):
        idx = dst_v[i]
        cnt, last = plsc.scan_count(idx)
        plsc.addupdate_scatter(tacc, [idx], cnt.astype(jnp.float32),
                               mask=last)
        return _

    lax.fori_loop(0, NE16, count, None)
    pltpu.sync_copy(tacc, out_hbm.at[wid])


@functools.partial(
    pl.kernel,
    out_type=jax.ShapeDtypeStruct((NC, NP, D), jnp.float32),
    mesh=_sc_mesh,
    compiler_params=pltpu.CompilerParams(use_tc_tiling_on_sc=False),
    scratch_types=[
        pltpu.VMEM((NCHUNK, CH), jnp.int32),
        pltpu.VMEM((NCHUNK, CH), jnp.int32),
        pltpu.VMEM((NB, CH, D), jnp.float32),
        pltpu.VMEM((80, D), jnp.float32),
        pltpu.SemaphoreType.DMA((NB,)),
        pltpu.SemaphoreType.DMA((NB,)),
        pltpu.SemaphoreType.DMA,
        pltpu.VMEM_SHARED((NP, D), jnp.float32),
    ],
)
def _sc_edge_pass(g_hbm, src_hbm, dst_hbm, out_hbm, src_v, dst_v, rows_v,
                  z_v, gsem, ssem, zsem, acc):
    cid = lax.axis_index("c")
    sid = lax.axis_index("s")
    wid = sid * NC + cid
    pltpu.sync_copy(src_hbm.at[wid], src_v)
    pltpu.sync_copy(dst_hbm.at[wid], dst_v)

    for b in range(NB):
        pltpu.async_copy(g_hbm.at[src_v.at[b]], rows_v.at[b], gsem.at[b])

    def zfill(i, _):
        for j in range(D // 16):
            z_v[i, pl.ds(j * 16, 16)] = jnp.zeros((16,), jnp.float32)
        return _

    lax.fori_loop(0, 80, zfill, None)
    for c in range(RPT // 80):
        pltpu.async_copy(z_v, acc.at[pl.ds(sid * RPT + c * 80, 80)], zsem)
    for c in range(RPT // 80):
        pltpu.make_async_copy(z_v, acc.at[pl.ds(sid * RPT, 80)], zsem).wait()
    plsc.subcore_barrier()

    def group(g, _):
        for b in range(NB):
            k = g * NB + b
            pltpu.make_async_copy(
                g_hbm.at[src_v.at[k]], rows_v.at[b], gsem.at[b]).wait()
            pltpu.async_copy(
                rows_v.at[b], acc.at[dst_v.at[k]], ssem.at[b], add=True)
            nk = k + NB

            @pl.when(nk < NCHUNK)
            def _start():
                pltpu.make_async_copy(
                    rows_v.at[b], acc.at[dst_v.at[k]], ssem.at[b]).wait()
                pltpu.async_copy(
                    g_hbm.at[src_v.at[nk]], rows_v.at[b], gsem.at[b])

        return _

    lax.fori_loop(0, NG, group, None)
    for b in range(NB):
        pltpu.make_async_copy(
            rows_v.at[b], acc.at[dst_v.at[0]], ssem.at[b]).wait()
    plsc.subcore_barrier()
    pltpu.sync_copy(acc.at[pl.ds(sid * RPT, RPT)],
                    out_hbm.at[cid, pl.ds(sid * RPT, RPT)])


_R = 2048


def _tc_matmul(x, w1):
    def body(x_ref, w_ref, u_ref):
        xr = x_ref[...].reshape(RH, 2, DIN)
        ue = jnp.dot(xr[:, 0, :], w_ref[...],
                     preferred_element_type=jnp.float32)
        uo = jnp.dot(xr[:, 1, :], w_ref[...],
                     preferred_element_type=jnp.float32)
        u_ref[...] = jnp.concatenate([ue, uo], axis=1)

    return pl.pallas_call(
        body,
        grid=(NP // _R,),
        in_specs=[
            pl.BlockSpec((_R, DIN), lambda i: (i, 0)),
            pl.BlockSpec((DIN, D), lambda i: (0, 0)),
        ],
        out_specs=pl.BlockSpec((RH, 2 * D), lambda i: (i, 0)),
        out_shape=jax.ShapeDtypeStruct((NH, 2 * D), jnp.float32),
    )(x, w1)


def _tc_mult_scale(degp, u):
    def body(dp_ref, u_ref, m_ref, g_ref):
        acc = dp_ref[0]
        for j in range(1, NW):
            acc = acc + dp_ref[j]
        me = lax.rsqrt(acc[:, 0:1] + 1.0)
        mo = lax.rsqrt(acc[:, 1:2] + 1.0)
        m = jnp.concatenate(
            [jnp.broadcast_to(me, (RH, D)), jnp.broadcast_to(mo, (RH, D))],
            axis=1)
        m_ref[...] = m
        g_ref[...] = m * u_ref[...]

    return pl.pallas_call(
        body,
        grid=(NH // RH,),
        in_specs=[
            pl.BlockSpec((NW, RH, 2), lambda i: (0, i, 0)),
            pl.BlockSpec((RH, 2 * D), lambda i: (i, 0)),
        ],
        out_specs=[
            pl.BlockSpec((RH, 2 * D), lambda i: (i, 0)),
            pl.BlockSpec((RH, 2 * D), lambda i: (i, 0)),
        ],
        out_shape=[
            jax.ShapeDtypeStruct((NH, 2 * D), jnp.float32),
            jax.ShapeDtypeStruct((NH, 2 * D), jnp.float32),
        ],
    )(degp.reshape(NW, NH, 2), u)


def _tc_mid(s, g, m, wbd, bp):
    def body(s_ref, g_ref, m_ref, w_ref, b_ref, o_ref):
        mm = m_ref[...]
        h = jnp.maximum(mm * (s_ref[0] + s_ref[1] + g_ref[...]) + b_ref[...],
                        0.0)
        o_ref[...] = mm * jnp.dot(h, w_ref[...],
                                  preferred_element_type=jnp.float32)

    return pl.pallas_call(
        body,
        grid=(NH // RH,),
        in_specs=[
            pl.BlockSpec((NC, RH, 2 * D), lambda i: (0, i, 0)),
            pl.BlockSpec((RH, 2 * D), lambda i: (i, 0)),
            pl.BlockSpec((RH, 2 * D), lambda i: (i, 0)),
            pl.BlockSpec((2 * D, 2 * D), lambda i: (0, 0)),
            pl.BlockSpec((1, 2 * D), lambda i: (0, 0)),
        ],
        out_specs=pl.BlockSpec((RH, 2 * D), lambda i: (i, 0)),
        out_shape=jax.ShapeDtypeStruct((NH, 2 * D), jnp.float32),
    )(s, g, m, wbd, bp)


def _tc_out(s, q, m, w2e, w2o, b2):
    def body(s_ref, q_ref, m_ref, we_ref, wo_ref, b_ref, o_ref):
        a = m_ref[...] * (s_ref[0] + s_ref[1] + q_ref[...])
        he = jnp.dot(a, we_ref[...], preferred_element_type=jnp.float32)
        ho = jnp.dot(a, wo_ref[...], preferred_element_type=jnp.float32)
        st = jnp.concatenate(
            [he.reshape(RH, 1, DOUT), ho.reshape(RH, 1, DOUT)], axis=1)
        o_ref[...] = jnp.maximum(st.reshape(2 * RH, DOUT) + b_ref[...], 0.0)

    return pl.pallas_call(
        body,
        grid=(NP // _R,),
        in_specs=[
            pl.BlockSpec((NC, RH, 2 * D), lambda i: (0, i, 0)),
            pl.BlockSpec((RH, 2 * D), lambda i: (i, 0)),
            pl.BlockSpec((RH, 2 * D), lambda i: (i, 0)),
            pl.BlockSpec((2 * D, DOUT), lambda i: (0, 0)),
            pl.BlockSpec((2 * D, DOUT), lambda i: (0, 0)),
            pl.BlockSpec((1, DOUT), lambda i: (0, 0)),
        ],
        out_specs=pl.BlockSpec((_R, DOUT), lambda i: (i, 0)),
        out_shape=jax.ShapeDtypeStruct((N, DOUT), jnp.float32),
    )(s, q, m, w2e, w2o, b2)


def kernel(x, edge_index, W1, b1, W3, b3, W2, b2):
    src = edge_index[0].reshape(NW, NCHUNK, CH)
    dst = edge_index[1].reshape(NW, NCHUNK, CH)
    u1 = _tc_matmul(x, W1)
    degp = _sc_degree(edge_index[1].reshape(NW, NE16, 16))
    m, g1 = _tc_mult_scale(degp, u1)

    z = jnp.zeros_like(W3)
    w3bd = jnp.block([[W3, z], [z, W3]])
    eye = jnp.eye(2 * D, dtype=jnp.float32)
    w2e = jnp.concatenate([W2, jnp.zeros_like(W2)], axis=0)
    w2o = jnp.concatenate([jnp.zeros_like(W2), W2], axis=0)
    bp1 = jnp.concatenate([b1, b1]).reshape(1, 2 * D)
    bp3 = jnp.concatenate([b3, b3]).reshape(1, 2 * D)

    def sc(gp):
        out = _sc_edge_pass(gp.reshape(NP, D), src, dst)
        return out.reshape(NC, NH, 2 * D)

    s1 = sc(g1)
    g2 = _tc_mid(s1, g1, m, w3bd, bp1)
    s2 = sc(g2)
    q = _tc_mid(s2, g2, m, eye, bp3)
    s3 = sc(q)
    return _tc_out(s3, q, m, w2e, w2o, b2.reshape(1, DOUT))
